# Initial kernel scaffold; baseline (speedup 1.0000x reference)
import jax
import jax.numpy as jnp
from jax import lax
from jax.experimental import pallas as pl
from jax.experimental.pallas import tpu as pltpu
from jax.experimental.pallas import tpu_sc as plsc

_KGE_DIM = 128
_DM = 768
_B = 16384

_info = plsc.get_sparse_core_info()
_NC, _NS = _info.num_cores, _info.num_subcores
_NW = _NC * _NS
_ROWS_PER_W = _B // _NW
_CHUNK = 128
_NCHUNK = _ROWS_PER_W // _CHUNK
_NJOBS = 3 * _NCHUNK


def _sc_gather_body(idx_hbm, ent_hbm, rel_hbm, out_hbm,
                    idx_v, rows_a, rows_b, sem_a, sem_b, sem_out):
    wid = lax.axis_index("s") * _NC + lax.axis_index("c")
    base = wid * _ROWS_PER_W
    for f in range(3):
        pltpu.sync_copy(idx_hbm.at[f, wid], idx_v.at[f])

    tables = (ent_hbm, rel_hbm, ent_hbm)
    bufs = (rows_a, rows_b)
    sems = (sem_a, sem_b)

    def gather(k):
        f, c = divmod(k, _NCHUNK)
        return pltpu.async_copy(
            tables[f].at[idx_v.at[f, c]], bufs[k % 2], sems[k % 2])

    def writeback(k):
        f, c = divmod(k, _NCHUNK)
        return pltpu.async_copy(
            bufs[k % 2], out_hbm.at[f, pl.ds(base + c * _CHUNK, _CHUNK)],
            sem_out)

    wdescs = [None] * _NJOBS
    gd = gather(0)
    for k in range(_NJOBS):
        gd.wait()
        wdescs[k] = writeback(k)
        if k + 1 < _NJOBS:
            if k >= 1:
                wdescs[k - 1].wait()
            gd = gather(k + 1)
    if _NJOBS >= 2:
        wdescs[_NJOBS - 2].wait()
    wdescs[_NJOBS - 1].wait()


def _sc_gather(idx_all, ent_emb, rel_emb):
    mesh = plsc.VectorSubcoreMesh(core_axis_name="c", subcore_axis_name="s")
    return pl.kernel(
        _sc_gather_body,
        out_type=jax.ShapeDtypeStruct((3, _B, _KGE_DIM), jnp.float32),
        mesh=mesh,
        scratch_types=[
            pltpu.VMEM((3, _NCHUNK, _CHUNK), jnp.int32),
            pltpu.VMEM((_CHUNK, _KGE_DIM), jnp.float32),
            pltpu.VMEM((_CHUNK, _KGE_DIM), jnp.float32),
            pltpu.SemaphoreType.DMA,
            pltpu.SemaphoreType.DMA,
            pltpu.SemaphoreType.DMA,
        ],
    )(idx_all, ent_emb, rel_emb)


_BM = 512


def _tc_matmul_body(emb_ref, w_ref, b_ref, out_ref):
    x = emb_ref[0]
    acc = jnp.dot(x, w_ref[...], preferred_element_type=jnp.float32)
    out_ref[:, 0, :] = acc + b_ref[...]


def _tc_matmul(embs, W, b):
    return pl.pallas_call(
        _tc_matmul_body,
        grid=(3, _B // _BM),
        in_specs=[
            pl.BlockSpec((1, _BM, _KGE_DIM), lambda f, i: (f, i, 0)),
            pl.BlockSpec((_KGE_DIM, _DM), lambda f, i: (0, 0)),
            pl.BlockSpec((_DM,), lambda f, i: (0,)),
        ],
        out_specs=pl.BlockSpec((_BM, 1, _DM), lambda f, i: (i, f, 0)),
        out_shape=jax.ShapeDtypeStruct((_B, 3, _DM), jnp.float32),
    )(embs, W, b)


def kernel(triples, ent_emb, rel_emb, W, b):
    idx_all = jnp.transpose(triples, (1, 0)).reshape(3, _NW, _NCHUNK, _CHUNK)
    embs = _sc_gather(idx_all, ent_emb, rel_emb)
    return _tc_matmul(embs, W, b)


# speedup vs baseline: 3.9464x; 3.9464x over previous
import jax
import jax.numpy as jnp
from jax import lax
from jax.experimental import pallas as pl
from jax.experimental.pallas import tpu as pltpu
from jax.experimental.pallas import tpu_sc as plsc

_KGE_DIM = 128
_DM = 768
_B = 16384

_info = plsc.get_sparse_core_info()
_NC, _NS = _info.num_cores, _info.num_subcores
_NW = _NC * _NS
_ROWS_PER_W = _B // _NW
_CHUNK = 128
_NCHUNK = _ROWS_PER_W // _CHUNK
_NJOBS = 3 * _NCHUNK


def _sc_gather_body(idx_hbm, ent_hbm, rel_hbm, out_hbm,
                    idx_v, rows_a, rows_b, sem_a, sem_b, sem_out):
    wid = lax.axis_index("s") * _NC + lax.axis_index("c")
    base = wid * _ROWS_PER_W
    for f in range(3):
        pltpu.sync_copy(idx_hbm.at[f, wid], idx_v.at[f])

    tables = (ent_hbm, rel_hbm, ent_hbm)
    bufs = (rows_a, rows_b)
    sems = (sem_a, sem_b)

    def gather(k):
        f, c = divmod(k, _NCHUNK)
        return pltpu.async_copy(
            tables[f].at[idx_v.at[f, c]], bufs[k % 2], sems[k % 2])

    def writeback(k):
        f, c = divmod(k, _NCHUNK)
        return pltpu.async_copy(
            bufs[k % 2], out_hbm.at[f, pl.ds(base + c * _CHUNK, _CHUNK)],
            sem_out)

    wdescs = [None] * _NJOBS
    gd = gather(0)
    for k in range(_NJOBS):
        gd.wait()
        wdescs[k] = writeback(k)
        if k + 1 < _NJOBS:
            if k >= 1:
                wdescs[k - 1].wait()
            gd = gather(k + 1)
    if _NJOBS >= 2:
        wdescs[_NJOBS - 2].wait()
    wdescs[_NJOBS - 1].wait()


def _sc_gather(idx_all, ent_emb, rel_emb):
    mesh = plsc.VectorSubcoreMesh(core_axis_name="c", subcore_axis_name="s")
    return pl.kernel(
        _sc_gather_body,
        out_type=jax.ShapeDtypeStruct((3, _B, _KGE_DIM), jnp.float32),
        mesh=mesh,
        scratch_types=[
            pltpu.VMEM((3, _NCHUNK, _CHUNK), jnp.int32),
            pltpu.VMEM((_CHUNK, _KGE_DIM), jnp.float32),
            pltpu.VMEM((_CHUNK, _KGE_DIM), jnp.float32),
            pltpu.SemaphoreType.DMA,
            pltpu.SemaphoreType.DMA,
            pltpu.SemaphoreType.DMA,
        ],
    )(idx_all, ent_emb, rel_emb)


_BM = 512


def _tc_matmul_body(emb_ref, w_ref, b_ref, out_ref):
    w = w_ref[...]
    bias = b_ref[...]
    for f in range(3):
        acc = jnp.dot(emb_ref[f], w, preferred_element_type=jnp.float32)
        out_ref[:, f, :] = acc + bias


def _tc_matmul(embs, W, b):
    return pl.pallas_call(
        _tc_matmul_body,
        grid=(_B // _BM,),
        in_specs=[
            pl.BlockSpec((3, _BM, _KGE_DIM), lambda i: (0, i, 0)),
            pl.BlockSpec((_KGE_DIM, _DM), lambda i: (0, 0)),
            pl.BlockSpec((_DM,), lambda i: (0,)),
        ],
        out_specs=pl.BlockSpec((_BM, 3, _DM), lambda i: (i, 0, 0)),
        out_shape=jax.ShapeDtypeStruct((_B, 3, _DM), jnp.float32),
    )(embs, W, b)


def kernel(triples, ent_emb, rel_emb, W, b):
    idx_all = jnp.transpose(triples, (1, 0)).reshape(3, _NW, _NCHUNK, _CHUNK)
    embs = _sc_gather(idx_all, ent_emb, rel_emb)
    return _tc_matmul(embs, W, b)


# speedup vs baseline: 4.0267x; 1.0204x over previous
import jax
import jax.numpy as jnp
from jax import lax
from jax.experimental import pallas as pl
from jax.experimental.pallas import tpu as pltpu
from jax.experimental.pallas import tpu_sc as plsc

_KGE_DIM = 128
_DM = 768
_B = 16384

_info = plsc.get_sparse_core_info()
_NC, _NS = _info.num_cores, _info.num_subcores
_NW = _NC * _NS
_ROWS_PER_W = _B // _NW
_CHUNK = 128
_NCHUNK = _ROWS_PER_W // _CHUNK
_NJOBS = 3 * _NCHUNK


def _sc_gather_body(idx_hbm, ent_hbm, rel_hbm, out_hbm,
                    idx_v, rows_a, rows_b, sem_a, sem_b, sem_out):
    wid = lax.axis_index("s") * _NC + lax.axis_index("c")
    base = wid * _ROWS_PER_W
    for f in range(3):
        pltpu.sync_copy(idx_hbm.at[f, wid], idx_v.at[f])

    tables = (ent_hbm, rel_hbm, ent_hbm)
    bufs = (rows_a, rows_b)
    sems = (sem_a, sem_b)

    def gather(k):
        f, c = divmod(k, _NCHUNK)
        return pltpu.async_copy(
            tables[f].at[idx_v.at[f, c]], bufs[k % 2], sems[k % 2])

    def writeback(k):
        f, c = divmod(k, _NCHUNK)
        return pltpu.async_copy(
            bufs[k % 2], out_hbm.at[f, pl.ds(base + c * _CHUNK, _CHUNK)],
            sem_out)

    wdescs = [None] * _NJOBS
    gd = gather(0)
    for k in range(_NJOBS):
        gd.wait()
        wdescs[k] = writeback(k)
        if k + 1 < _NJOBS:
            if k >= 1:
                wdescs[k - 1].wait()
            gd = gather(k + 1)
    if _NJOBS >= 2:
        wdescs[_NJOBS - 2].wait()
    wdescs[_NJOBS - 1].wait()


def _sc_gather(idx_all, ent_emb, rel_emb):
    mesh = plsc.VectorSubcoreMesh(core_axis_name="c", subcore_axis_name="s")
    return pl.kernel(
        _sc_gather_body,
        out_type=jax.ShapeDtypeStruct((3, _B, _KGE_DIM), jnp.float32),
        mesh=mesh,
        scratch_types=[
            pltpu.VMEM((3, _NCHUNK, _CHUNK), jnp.int32),
            pltpu.VMEM((_CHUNK, _KGE_DIM), jnp.float32),
            pltpu.VMEM((_CHUNK, _KGE_DIM), jnp.float32),
            pltpu.SemaphoreType.DMA,
            pltpu.SemaphoreType.DMA,
            pltpu.SemaphoreType.DMA,
        ],
    )(idx_all, ent_emb, rel_emb)


_BM = 1024


def _tc_matmul_body(emb_ref, w_ref, b_ref, out_ref):
    w = w_ref[...]
    bias = b_ref[...]
    for f in range(3):
        acc = jnp.dot(emb_ref[f], w, preferred_element_type=jnp.float32)
        out_ref[:, f, :] = acc + bias


def _tc_matmul(embs, W, b):
    return pl.pallas_call(
        _tc_matmul_body,
        grid=(_B // _BM,),
        in_specs=[
            pl.BlockSpec((3, _BM, _KGE_DIM), lambda i: (0, i, 0)),
            pl.BlockSpec((_KGE_DIM, _DM), lambda i: (0, 0)),
            pl.BlockSpec((_DM,), lambda i: (0,)),
        ],
        out_specs=pl.BlockSpec((_BM, 3, _DM), lambda i: (i, 0, 0)),
        out_shape=jax.ShapeDtypeStruct((_B, 3, _DM), jnp.float32),
    )(embs, W, b)


def kernel(triples, ent_emb, rel_emb, W, b):
    idx_all = jnp.transpose(triples, (1, 0)).reshape(3, _NW, _NCHUNK, _CHUNK)
    embs = _sc_gather(idx_all, ent_emb, rel_emb)
    return _tc_matmul(embs, W, b)


# speedup vs baseline: 9.0828x; 2.2556x over previous
import jax
import jax.numpy as jnp
from jax import lax
from jax.experimental import pallas as pl
from jax.experimental.pallas import tpu as pltpu
from jax.experimental.pallas import tpu_sc as plsc

_KGE_DIM = 128
_DM = 768
_B = 16384

_info = plsc.get_sparse_core_info()
_NC, _NS = _info.num_cores, _info.num_subcores
_NW = _NC * _NS
_ROWS_PER_W = _B // _NW
_CHUNK = 128
_NCHUNK = _ROWS_PER_W // _CHUNK
_NJOBS = 3 * _NCHUNK


def _sc_gather_body(idx_hbm, ent_hbm, rel_hbm, out_hbm,
                    idx_v, rows_a, rows_b, sem_a, sem_b, sem_out):
    wid = lax.axis_index("s") * _NC + lax.axis_index("c")
    base = wid * _ROWS_PER_W
    for f in range(3):
        pltpu.sync_copy(idx_hbm.at[f, wid], idx_v.at[f])

    tables = (ent_hbm, rel_hbm, ent_hbm)
    bufs = (rows_a, rows_b)
    sems = (sem_a, sem_b)

    def gather(k):
        f, c = divmod(k, _NCHUNK)
        return pltpu.async_copy(
            tables[f].at[idx_v.at[f, c]], bufs[k % 2], sems[k % 2])

    def writeback(k):
        f, c = divmod(k, _NCHUNK)
        return pltpu.async_copy(
            bufs[k % 2], out_hbm.at[f, pl.ds(base + c * _CHUNK, _CHUNK)],
            sem_out)

    wdescs = [None] * _NJOBS
    gd = gather(0)
    for k in range(_NJOBS):
        gd.wait()
        wdescs[k] = writeback(k)
        if k + 1 < _NJOBS:
            if k >= 1:
                wdescs[k - 1].wait()
            gd = gather(k + 1)
    if _NJOBS >= 2:
        wdescs[_NJOBS - 2].wait()
    wdescs[_NJOBS - 1].wait()


def _sc_gather(idx_all, ent_emb, rel_emb):
    mesh = plsc.VectorSubcoreMesh(core_axis_name="c", subcore_axis_name="s")
    return pl.kernel(
        _sc_gather_body,
        out_type=jax.ShapeDtypeStruct((3, _B, _KGE_DIM), jnp.float32),
        mesh=mesh,
        scratch_types=[
            pltpu.VMEM((3, _NCHUNK, _CHUNK), jnp.int32),
            pltpu.VMEM((_CHUNK, _KGE_DIM), jnp.float32),
            pltpu.VMEM((_CHUNK, _KGE_DIM), jnp.float32),
            pltpu.SemaphoreType.DMA,
            pltpu.SemaphoreType.DMA,
            pltpu.SemaphoreType.DMA,
        ],
    )(idx_all, ent_emb, rel_emb)


_BM = 1024


def _tc_matmul_body(emb_ref, w_ref, b_ref, out_ref):
    w = w_ref[...]
    bias = b_ref[...]
    for f in range(3):
        acc = jnp.dot(emb_ref[f], w, preferred_element_type=jnp.float32)
        out_ref[f] = acc + bias


def _tc_matmul(embs, W, b):
    return pl.pallas_call(
        _tc_matmul_body,
        grid=(_B // _BM,),
        in_specs=[
            pl.BlockSpec((3, _BM, _KGE_DIM), lambda i: (0, i, 0)),
            pl.BlockSpec((_KGE_DIM, _DM), lambda i: (0, 0)),
            pl.BlockSpec((_DM,), lambda i: (0,)),
        ],
        out_specs=pl.BlockSpec((3, _BM, _DM), lambda i: (0, i, 0)),
        out_shape=jax.ShapeDtypeStruct((3, _B, _DM), jnp.float32),
    )(embs, W, b)


def kernel(triples, ent_emb, rel_emb, W, b):
    idx_all = jnp.transpose(triples, (1, 0)).reshape(3, _NW, _NCHUNK, _CHUNK)
    embs = _sc_gather(idx_all, ent_emb, rel_emb)
    out = _tc_matmul(embs, W, b)
    return jnp.transpose(out, (1, 0, 2))


# speedup vs baseline: 9.5867x; 1.0555x over previous
import jax
import jax.numpy as jnp
from jax import lax
from jax.experimental import pallas as pl
from jax.experimental.pallas import tpu as pltpu
from jax.experimental.pallas import tpu_sc as plsc

_KGE_DIM = 128
_DM = 768
_B = 16384

_info = plsc.get_sparse_core_info()
_NC, _NS = _info.num_cores, _info.num_subcores
_NW = _NC * _NS
_ROWS_PER_W = _B // _NW
_CHUNK = 128
_NCHUNK = _ROWS_PER_W // _CHUNK
_NJOBS = 3 * _NCHUNK


_NBUF = 4


def _sc_gather_body(idx_hbm, ent_hbm, rel_hbm, out_hbm,
                    idx_v, rows_v, gsems, wsems):
    wid = lax.axis_index("s") * _NC + lax.axis_index("c")
    base = wid * _ROWS_PER_W
    for f in range(3):
        pltpu.sync_copy(idx_hbm.at[f, wid], idx_v.at[f])

    tables = (ent_hbm, rel_hbm, ent_hbm)

    def gather(k):
        f, c = divmod(k, _NCHUNK)
        j = k % _NBUF
        return pltpu.async_copy(
            tables[f].at[idx_v.at[f, c]], rows_v.at[j], gsems[j])

    def writeback(k):
        f, c = divmod(k, _NCHUNK)
        j = k % _NBUF
        return pltpu.async_copy(
            rows_v.at[j], out_hbm.at[f, pl.ds(base + c * _CHUNK, _CHUNK)],
            wsems[j])

    gdescs = [None] * _NJOBS
    wdescs = [None] * _NJOBS
    for k in range(min(_NBUF, _NJOBS)):
        gdescs[k] = gather(k)
    for k in range(_NJOBS):
        gdescs[k].wait()
        wdescs[k] = writeback(k)
        nxt = k + _NBUF
        if nxt < _NJOBS:
            wdescs[k].wait()
            gdescs[nxt] = gather(nxt)
    for k in range(max(0, _NJOBS - _NBUF), _NJOBS):
        wdescs[k].wait()


def _sc_gather(idx_all, ent_emb, rel_emb):
    mesh = plsc.VectorSubcoreMesh(core_axis_name="c", subcore_axis_name="s")
    return pl.kernel(
        _sc_gather_body,
        out_type=jax.ShapeDtypeStruct((3, _B, _KGE_DIM), jnp.float32),
        mesh=mesh,
        scratch_types=[
            pltpu.VMEM((3, _NCHUNK, _CHUNK), jnp.int32),
            pltpu.VMEM((_NBUF, _CHUNK, _KGE_DIM), jnp.float32),
            [pltpu.SemaphoreType.DMA] * _NBUF,
            [pltpu.SemaphoreType.DMA] * _NBUF,
        ],
    )(idx_all, ent_emb, rel_emb)


_BM = 1024


def _tc_matmul_body(emb_ref, w_ref, b_ref, out_ref):
    w = w_ref[...]
    bias = b_ref[...]
    for f in range(3):
        acc = jnp.dot(emb_ref[f], w, preferred_element_type=jnp.float32)
        out_ref[f] = acc + bias


def _tc_matmul(embs, W, b):
    return pl.pallas_call(
        _tc_matmul_body,
        grid=(_B // _BM,),
        in_specs=[
            pl.BlockSpec((3, _BM, _KGE_DIM), lambda i: (0, i, 0)),
            pl.BlockSpec((_KGE_DIM, _DM), lambda i: (0, 0)),
            pl.BlockSpec((_DM,), lambda i: (0,)),
        ],
        out_specs=pl.BlockSpec((3, _BM, _DM), lambda i: (0, i, 0)),
        out_shape=jax.ShapeDtypeStruct((3, _B, _DM), jnp.float32),
    )(embs, W, b)


def kernel(triples, ent_emb, rel_emb, W, b):
    idx_all = jnp.transpose(triples, (1, 0)).reshape(3, _NW, _NCHUNK, _CHUNK)
    embs = _sc_gather(idx_all, ent_emb, rel_emb)
    out = _tc_matmul(embs, W, b)
    return jnp.transpose(out, (1, 0, 2))
